# baseline (speedup 1.0000x reference)
import functools

import jax
import jax.numpy as jnp
from jax import lax
from jax.experimental import pallas as pl
from jax.experimental.pallas import tpu as pltpu
from jax.experimental.pallas import tpu_sc as plsc

NC = 2
NS = 16
L = 16
NW = NC * NS
BATCH = 16384
DIM = 64
RPW = BATCH // NW
CHUNK = 128
NCHUNK = RPW // CHUNK

_mesh = plsc.VectorSubcoreMesh(
    core_axis_name="c", subcore_axis_name="s", num_cores=NC, num_subcores=NS
)


@functools.partial(
    pl.kernel,
    out_type=jax.ShapeDtypeStruct((BATCH,), jnp.float32),
    mesh=_mesh,
    compiler_params=pltpu.CompilerParams(
        needs_layout_passes=False, use_tc_tiling_on_sc=False
    ),
    scratch_types=[
        pltpu.VMEM((NCHUNK, CHUNK), jnp.int32),
        pltpu.VMEM((NCHUNK, CHUNK), jnp.int32),
        pltpu.VMEM((RPW, DIM), jnp.float32),
        pltpu.VMEM((RPW, DIM), jnp.float32),
        pltpu.VMEM((RPW,), jnp.float32),
        pltpu.SemaphoreType.DMA,
        pltpu.SemaphoreType.DMA,
    ],
)
def _bprmf_sc(u_hbm, i_hbm, uw_hbm, iw_hbm, out_hbm,
              uidx, iidx, urows, irows, outv, sem_u, sem_i):
    wid = lax.axis_index("s") * NC + lax.axis_index("c")
    base = wid * RPW

    for c in range(NCHUNK):
        pltpu.sync_copy(u_hbm.at[pl.ds(base + c * CHUNK, CHUNK)], uidx.at[c])
        pltpu.sync_copy(i_hbm.at[pl.ds(base + c * CHUNK, CHUNK)], iidx.at[c])

    copies = []
    for c in range(NCHUNK):
        dst = pl.ds(c * CHUNK, CHUNK)
        copies.append(pltpu.async_copy(uw_hbm.at[uidx.at[c]], urows.at[dst], sem_u))
        copies.append(pltpu.async_copy(iw_hbm.at[iidx.at[c]], irows.at[dst], sem_i))
    for cp in copies:
        cp.wait()

    def group(g, carry):
        rows = g * L + lax.iota(jnp.int32, L)
        acc = jnp.zeros((L,), jnp.float32)
        for d in range(DIM):
            dv = jnp.full((L,), d, jnp.int32)
            uv = plsc.load_gather(urows, [rows, dv])
            iv = plsc.load_gather(irows, [rows, dv])
            acc = acc + uv * iv
        outv[pl.ds(g * L, L)] = acc
        return carry

    lax.fori_loop(0, RPW // L, group, 0)

    pltpu.sync_copy(outv, out_hbm.at[pl.ds(base, RPW)])


def kernel(u, i, user_weight, item_weight):
    return _bprmf_sc(u.astype(jnp.int32), i.astype(jnp.int32),
                     user_weight, item_weight)


# speedup vs baseline: 2.1221x; 2.1221x over previous
import functools

import jax
import jax.numpy as jnp
from jax import lax
from jax.experimental import pallas as pl
from jax.experimental.pallas import tpu as pltpu
from jax.experimental.pallas import tpu_sc as plsc

NC = 2
NS = 16
L = 16
NW = NC * NS
BATCH = 16384
DIM = 64
BLK = 8
NBLOCKS = 1000000 // BLK
RPW = BATCH // NW
CHUNK = 32
NCHUNK = RPW // CHUNK

_mesh = plsc.VectorSubcoreMesh(
    core_axis_name="c", subcore_axis_name="s", num_cores=NC, num_subcores=NS
)


@functools.partial(
    pl.kernel,
    out_type=jax.ShapeDtypeStruct((BATCH,), jnp.float32),
    mesh=_mesh,
    compiler_params=pltpu.CompilerParams(needs_layout_passes=False),
    scratch_types=[
        pltpu.VMEM((RPW,), jnp.int32),
        pltpu.VMEM((RPW,), jnp.int32),
        pltpu.VMEM((CHUNK, BLK, DIM), jnp.float32),
        pltpu.VMEM((CHUNK, BLK, DIM), jnp.float32),
        pltpu.VMEM((RPW,), jnp.float32),
        pltpu.SemaphoreType.DMA,
        pltpu.SemaphoreType.DMA,
    ],
)
def _bprmf_sc(u_hbm, i_hbm, uw_hbm, iw_hbm, out_hbm,
              uraw, iraw, ublk, iblk, outv, sem_u, sem_i):
    wid = lax.axis_index("s") * NC + lax.axis_index("c")
    base = wid * RPW

    pltpu.sync_copy(u_hbm.at[pl.ds(base, RPW)], uraw)
    pltpu.sync_copy(i_hbm.at[pl.ds(base, RPW)], iraw)

    def chunk_body(c, carry):
        cps = []
        for g in range(CHUNK // L):
            sl = pl.ds(c * CHUNK + g * L, L)
            ubv = lax.shift_right_logical(uraw[sl], 3)
            ibv = lax.shift_right_logical(iraw[sl], 3)
            for s in range(L):
                slot = g * L + s
                cps.append(pltpu.async_copy(
                    uw_hbm.at[ubv[s]], ublk.at[slot], sem_u))
                cps.append(pltpu.async_copy(
                    iw_hbm.at[ibv[s]], iblk.at[slot], sem_i))
        for cp in cps:
            cp.wait()
        for g in range(CHUNK // L):
            sl = pl.ds(c * CHUNK + g * L, L)
            ur = jnp.bitwise_and(uraw[sl], 7)
            ir = jnp.bitwise_and(iraw[sl], 7)
            gslots = lax.iota(jnp.int32, L) + g * L
            acc = jnp.zeros((L,), jnp.float32)
            for f in range(DIM):
                fv = jnp.full((L,), f, jnp.int32)
                uv = plsc.load_gather(ublk, [gslots, ur, fv])
                iv = plsc.load_gather(iblk, [gslots, ir, fv])
                acc = acc + uv * iv
            outv[pl.ds(c * CHUNK + g * L, L)] = acc
        return carry

    lax.fori_loop(0, NCHUNK, chunk_body, 0)

    pltpu.sync_copy(outv, out_hbm.at[pl.ds(base, RPW)])


def kernel(u, i, user_weight, item_weight):
    uw3 = jnp.reshape(user_weight, (NBLOCKS, BLK, DIM))
    iw3 = jnp.reshape(item_weight, (NBLOCKS, BLK, DIM))
    return _bprmf_sc(u.astype(jnp.int32), i.astype(jnp.int32), uw3, iw3)
